# baseline (speedup 1.0000x reference)
import functools
import math

import jax
import jax.numpy as jnp
from jax.experimental import pallas as pl
from jax.experimental.pallas import tpu as pltpu

SAMPLE_MERGE_TEMP = 1.0
KB = 16


def _proposal_kernel(n_ref, emb_ref, gum_ref, leaf_ref, w1_ref, b1_ref,
                     w2_ref, b2_ref,
                     idx1_ref, idx2_ref, br1_ref, br2_ref, emb_out_ref,
                     lvp_ref, lvm_ref, *, kb, t, d):
    f32 = jnp.float32
    e = emb_ref[...]

    inners = jnp.stack(
        [jax.lax.dot_general(e[i], e[i], (((1,), (1,)), ((), ())),
                             preferred_element_type=f32)
         for i in range(kb)], axis=0)

    sq = jnp.sum(e * e, axis=2)
    d2 = sq[:, :, None] + sq[:, None, :] - 2.0 * inners
    dist = jnp.sqrt(jnp.maximum(d2, 1e-12))

    row = jax.lax.broadcasted_iota(jnp.int32, (kb, t, t), 1)
    col = jax.lax.broadcasted_iota(jnp.int32, (kb, t, t), 2)
    eye = row == col
    neg_inf = jnp.float32(-jnp.inf)
    logits = jnp.where(eye, neg_inf, -dist / SAMPLE_MERGE_TEMP)

    score = logits + gum_ref[...]
    m_s = jnp.max(score, axis=(1, 2), keepdims=True)
    flat_iota = row * t + col
    flat = jnp.min(jnp.where(score == m_s, flat_iota, t * t),
                   axis=(1, 2), keepdims=True)
    idx1 = flat // t
    idx2 = flat % t

    m_l = jnp.max(logits, axis=(1, 2), keepdims=True)
    lse = jnp.log(jnp.sum(jnp.exp(logits - m_l), axis=(1, 2))) + m_l[:, 0, 0]

    sel_logit = jnp.sum(jnp.where(flat_iota == flat, logits, 0.0), axis=(1, 2))
    log_v_plus = sel_logit + jnp.log(2.0) - lse

    sub = jax.lax.broadcasted_iota(jnp.int32, (kb, t, 1), 1)
    c1 = jnp.sum(e * (sub == idx1).astype(f32), axis=1)
    c2 = jnp.sum(e * (sub == idx2).astype(f32), axis=1)

    cat = jnp.concatenate([c1, c2], axis=1)
    h = jnp.dot(cat, w1_ref[...], preferred_element_type=f32) + b1_ref[...]
    h = jnp.maximum(h, 0.0)
    m = jnp.dot(h, w2_ref[...], preferred_element_type=f32) + b2_ref[...]

    br1 = jnp.sqrt(jnp.maximum(jnp.sum((c1 - m) ** 2, axis=1), 1e-12))
    br2 = jnp.sqrt(jnp.maximum(jnp.sum((c2 - m) ** 2, axis=1), 1e-12))

    lc = leaf_ref[...]
    sub2 = jax.lax.broadcasted_iota(jnp.int32, (kb, t), 1)
    i1 = idx1[:, :, 0]
    i2 = idx2[:, :, 0]
    l1 = jnp.sum(jnp.where(sub2 == i1, lc, 0), axis=1)
    l2 = jnp.sum(jnp.where(sub2 == i2, lc, 0), axis=1)
    none1 = jnp.sum((lc == 1).astype(jnp.int32), axis=1)
    none1 = none1 - (l1 == 1).astype(jnp.int32) - (l2 == 1).astype(jnp.int32)
    v_minus = n_ref[0, 0] - none1
    log_v_minus = jnp.log(v_minus.astype(f32))

    idx1_ref[...] = idx1[:, 0, 0].reshape(1, 1, kb)
    idx2_ref[...] = idx2[:, 0, 0].reshape(1, 1, kb)
    br1_ref[...] = br1.reshape(1, 1, kb)
    br2_ref[...] = br2.reshape(1, 1, kb)
    emb_out_ref[...] = m
    lvp_ref[...] = log_v_plus.reshape(1, 1, kb)
    lvm_ref[...] = log_v_minus.reshape(1, 1, kb)


def kernel(N, leaf_counts_Kxt, embeddings_KxtxD, log, W1, b1, W2, b2):
    k, t, d = embeddings_KxtxD.shape
    h = W1.shape[1]
    kb = KB
    nb = k // kb

    gum = jax.random.gumbel(jax.random.key(42), (k, t * t), jnp.float32)
    gum = gum.reshape(k, t, t)
    n_arr = jnp.asarray(N, jnp.int32).reshape(1, 1)
    leaf_i32 = leaf_counts_Kxt.astype(jnp.int32)
    b1_2d = b1.reshape(1, h)
    b2_2d = b2.reshape(1, d)

    vec_shape = jax.ShapeDtypeStruct((nb, 1, kb), jnp.float32)
    int_shape = jax.ShapeDtypeStruct((nb, 1, kb), jnp.int32)
    out_shapes = (
        int_shape, int_shape, vec_shape, vec_shape,
        jax.ShapeDtypeStruct((k, d), jnp.float32),
        vec_shape, vec_shape,
    )
    vec_spec = pl.BlockSpec((1, 1, kb), lambda i: (i, 0, 0))
    out_specs = (
        vec_spec, vec_spec, vec_spec, vec_spec,
        pl.BlockSpec((kb, d), lambda i: (i, 0)),
        vec_spec, vec_spec,
    )
    in_specs = (
        pl.BlockSpec((1, 1), lambda i: (0, 0)),
        pl.BlockSpec((kb, t, d), lambda i: (i, 0, 0)),
        pl.BlockSpec((kb, t, t), lambda i: (i, 0, 0)),
        pl.BlockSpec((kb, t), lambda i: (i, 0)),
        pl.BlockSpec((2 * d, h), lambda i: (0, 0)),
        pl.BlockSpec((1, h), lambda i: (0, 0)),
        pl.BlockSpec((h, d), lambda i: (0, 0)),
        pl.BlockSpec((1, d), lambda i: (0, 0)),
    )

    outs = pl.pallas_call(
        functools.partial(_proposal_kernel, kb=kb, t=t, d=d),
        grid=(nb,),
        in_specs=in_specs,
        out_specs=out_specs,
        out_shape=out_shapes,
        compiler_params=pltpu.CompilerParams(
            dimension_semantics=("arbitrary",)),
    )(n_arr, embeddings_KxtxD, gum, leaf_i32, W1, b1_2d, W2, b2_2d)

    idx1, idx2, br1, br2, emb_out, lvp, lvm = outs
    return (idx1.reshape(k), idx2.reshape(k), br1.reshape(k), br2.reshape(k),
            emb_out, lvp.reshape(k), lvm.reshape(k))


# speedup vs baseline: 1.0156x; 1.0156x over previous
import functools
import math

import jax
import jax.numpy as jnp
from jax.experimental import pallas as pl
from jax.experimental.pallas import tpu as pltpu

SAMPLE_MERGE_TEMP = 1.0
KB = 16

_GUMBEL_CACHE = {}


def _gumbel_const(k, t):
    if (k, t) not in _GUMBEL_CACHE:
        g = jax.random.gumbel(jax.random.key(42), (k, t * t), jnp.float32)
        _GUMBEL_CACHE[(k, t)] = g.reshape(k, t, t)
    return _GUMBEL_CACHE[(k, t)]


def _proposal_kernel(n_ref, emb_ref, gum_ref, leaf_ref, w1_ref, b1_ref,
                     w2_ref, b2_ref,
                     idx1_ref, idx2_ref, br1_ref, br2_ref, emb_out_ref,
                     lvp_ref, lvm_ref, *, kb, t, d):
    f32 = jnp.float32
    e = emb_ref[...]

    inners = jnp.stack(
        [jax.lax.dot_general(e[i], e[i], (((1,), (1,)), ((), ())),
                             preferred_element_type=f32)
         for i in range(kb)], axis=0)

    sq = jnp.sum(e * e, axis=2)
    d2 = sq[:, :, None] + sq[:, None, :] - 2.0 * inners
    dist = jnp.sqrt(jnp.maximum(d2, 1e-12))

    row = jax.lax.broadcasted_iota(jnp.int32, (kb, t, t), 1)
    col = jax.lax.broadcasted_iota(jnp.int32, (kb, t, t), 2)
    distm = jnp.where(row == col, jnp.float32(jnp.inf), dist)

    score = gum_ref[...] - distm
    m_s = jnp.max(score, axis=(1, 2), keepdims=True)
    flat_iota = row * t + col
    flat = jnp.min(jnp.where(score == m_s, flat_iota, t * t),
                   axis=(1, 2), keepdims=True)
    idx1 = flat // t
    idx2 = flat % t

    dmin = jnp.min(distm, axis=(1, 2), keepdims=True)
    lse = jnp.log(jnp.sum(jnp.exp(dmin - distm), axis=(1, 2))) - dmin[:, 0, 0]

    e_flat = e.reshape(kb * t, d)
    base = jax.lax.broadcasted_iota(jnp.int32, (2 * kb, 1), 0) % kb * t
    idx_cat = jnp.concatenate([idx1[:, 0, :], idx2[:, 0, :]], axis=0)
    pos = base + idx_cat
    onehot = (jax.lax.broadcasted_iota(jnp.int32, (2 * kb, kb * t), 1)
              == pos).astype(f32)
    cc = jax.lax.dot_general(onehot, e_flat, (((1,), (0,)), ((), ())),
                             preferred_element_type=f32)
    c1 = cc[:kb]
    c2 = cc[kb:]

    sel_logit = -jnp.sqrt(jnp.maximum(jnp.sum((c1 - c2) ** 2, axis=1), 1e-12))
    log_v_plus = sel_logit + jnp.log(2.0) - lse

    cat = jnp.concatenate([c1, c2], axis=1)
    h = jnp.dot(cat, w1_ref[...], preferred_element_type=f32) + b1_ref[...]
    h = jnp.maximum(h, 0.0)
    m = jnp.dot(h, w2_ref[...], preferred_element_type=f32) + b2_ref[...]

    br1 = jnp.sqrt(jnp.maximum(jnp.sum((c1 - m) ** 2, axis=1), 1e-12))
    br2 = jnp.sqrt(jnp.maximum(jnp.sum((c2 - m) ** 2, axis=1), 1e-12))

    lc = leaf_ref[...]
    sub2 = jax.lax.broadcasted_iota(jnp.int32, (kb, t), 1)
    i1 = idx1[:, :, 0]
    i2 = idx2[:, :, 0]
    l1 = jnp.sum(jnp.where(sub2 == i1, lc, 0), axis=1)
    l2 = jnp.sum(jnp.where(sub2 == i2, lc, 0), axis=1)
    none1 = jnp.sum((lc == 1).astype(jnp.int32), axis=1)
    none1 = none1 - (l1 == 1).astype(jnp.int32) - (l2 == 1).astype(jnp.int32)
    v_minus = n_ref[0, 0] - none1
    log_v_minus = jnp.log(v_minus.astype(f32))

    idx1_ref[...] = idx1[:, 0, 0].reshape(1, 1, kb)
    idx2_ref[...] = idx2[:, 0, 0].reshape(1, 1, kb)
    br1_ref[...] = br1.reshape(1, 1, kb)
    br2_ref[...] = br2.reshape(1, 1, kb)
    emb_out_ref[...] = m
    lvp_ref[...] = log_v_plus.reshape(1, 1, kb)
    lvm_ref[...] = log_v_minus.reshape(1, 1, kb)


def kernel(N, leaf_counts_Kxt, embeddings_KxtxD, log, W1, b1, W2, b2):
    k, t, d = embeddings_KxtxD.shape
    h = W1.shape[1]
    kb = KB
    nb = k // kb

    gum = _gumbel_const(k, t)
    n_arr = jnp.asarray(N, jnp.int32).reshape(1, 1)
    leaf_i32 = leaf_counts_Kxt.astype(jnp.int32)
    b1_2d = b1.reshape(1, h)
    b2_2d = b2.reshape(1, d)

    vec_shape = jax.ShapeDtypeStruct((nb, 1, kb), jnp.float32)
    int_shape = jax.ShapeDtypeStruct((nb, 1, kb), jnp.int32)
    out_shapes = (
        int_shape, int_shape, vec_shape, vec_shape,
        jax.ShapeDtypeStruct((k, d), jnp.float32),
        vec_shape, vec_shape,
    )
    vec_spec = pl.BlockSpec((1, 1, kb), lambda i: (i, 0, 0))
    out_specs = (
        vec_spec, vec_spec, vec_spec, vec_spec,
        pl.BlockSpec((kb, d), lambda i: (i, 0)),
        vec_spec, vec_spec,
    )
    in_specs = (
        pl.BlockSpec((1, 1), lambda i: (0, 0)),
        pl.BlockSpec((kb, t, d), lambda i: (i, 0, 0)),
        pl.BlockSpec((kb, t, t), lambda i: (i, 0, 0)),
        pl.BlockSpec((kb, t), lambda i: (i, 0)),
        pl.BlockSpec((2 * d, h), lambda i: (0, 0)),
        pl.BlockSpec((1, h), lambda i: (0, 0)),
        pl.BlockSpec((h, d), lambda i: (0, 0)),
        pl.BlockSpec((1, d), lambda i: (0, 0)),
    )

    outs = pl.pallas_call(
        functools.partial(_proposal_kernel, kb=kb, t=t, d=d),
        grid=(nb,),
        in_specs=in_specs,
        out_specs=out_specs,
        out_shape=out_shapes,
        compiler_params=pltpu.CompilerParams(
            dimension_semantics=("arbitrary",)),
    )(n_arr, embeddings_KxtxD, gum, leaf_i32, W1, b1_2d, W2, b2_2d)

    idx1, idx2, br1, br2, emb_out, lvp, lvm = outs
    return (idx1.reshape(k), idx2.reshape(k), br1.reshape(k), br2.reshape(k),
            emb_out, lvp.reshape(k), lvm.reshape(k))


# speedup vs baseline: 2.3374x; 2.3014x over previous
import functools
import math

import jax
import jax.numpy as jnp
from jax.experimental import pallas as pl
from jax.experimental.pallas import tpu as pltpu

SAMPLE_MERGE_TEMP = 1.0
KB = 16

_GUMBEL_CACHE = {}


def _gumbel_const(k, t):
    if (k, t) not in _GUMBEL_CACHE:
        with jax.ensure_compile_time_eval():
            g = jax.random.gumbel(jax.random.key(42), (k, t * t), jnp.float32)
            _GUMBEL_CACHE[(k, t)] = jax.block_until_ready(g.reshape(k, t, t))
    return _GUMBEL_CACHE[(k, t)]


def _proposal_kernel(n_ref, emb_ref, gum_ref, leaf_ref, w1_ref, b1_ref,
                     w2_ref, b2_ref,
                     idx1_ref, idx2_ref, br1_ref, br2_ref, emb_out_ref,
                     lvp_ref, lvm_ref, *, kb, t, d):
    f32 = jnp.float32
    e = emb_ref[...]

    inners = jnp.stack(
        [jax.lax.dot_general(e[i], e[i], (((1,), (1,)), ((), ())),
                             preferred_element_type=f32)
         for i in range(kb)], axis=0)

    sq = jnp.sum(e * e, axis=2)
    d2 = sq[:, :, None] + sq[:, None, :] - 2.0 * inners
    dist = jnp.sqrt(jnp.maximum(d2, 1e-12))

    row = jax.lax.broadcasted_iota(jnp.int32, (kb, t, t), 1)
    col = jax.lax.broadcasted_iota(jnp.int32, (kb, t, t), 2)
    distm = jnp.where(row == col, jnp.float32(jnp.inf), dist)

    score = gum_ref[...] - distm
    m_s = jnp.max(score, axis=(1, 2), keepdims=True)
    flat_iota = row * t + col
    flat = jnp.min(jnp.where(score == m_s, flat_iota, t * t),
                   axis=(1, 2), keepdims=True)
    idx1 = flat // t
    idx2 = flat % t

    dmin = jnp.min(distm, axis=(1, 2), keepdims=True)
    lse = jnp.log(jnp.sum(jnp.exp(dmin - distm), axis=(1, 2))) - dmin[:, 0, 0]

    e_flat = e.reshape(kb * t, d)
    base = jax.lax.broadcasted_iota(jnp.int32, (2 * kb, 1), 0) % kb * t
    idx_cat = jnp.concatenate([idx1[:, 0, :], idx2[:, 0, :]], axis=0)
    pos = base + idx_cat
    onehot = (jax.lax.broadcasted_iota(jnp.int32, (2 * kb, kb * t), 1)
              == pos).astype(f32)
    cc = jax.lax.dot_general(onehot, e_flat, (((1,), (0,)), ((), ())),
                             preferred_element_type=f32)
    c1 = cc[:kb]
    c2 = cc[kb:]

    sel_logit = -jnp.sqrt(jnp.maximum(jnp.sum((c1 - c2) ** 2, axis=1), 1e-12))
    log_v_plus = sel_logit + jnp.log(2.0) - lse

    cat = jnp.concatenate([c1, c2], axis=1)
    h = jnp.dot(cat, w1_ref[...], preferred_element_type=f32) + b1_ref[...]
    h = jnp.maximum(h, 0.0)
    m = jnp.dot(h, w2_ref[...], preferred_element_type=f32) + b2_ref[...]

    br1 = jnp.sqrt(jnp.maximum(jnp.sum((c1 - m) ** 2, axis=1), 1e-12))
    br2 = jnp.sqrt(jnp.maximum(jnp.sum((c2 - m) ** 2, axis=1), 1e-12))

    lc = leaf_ref[...]
    sub2 = jax.lax.broadcasted_iota(jnp.int32, (kb, t), 1)
    i1 = idx1[:, :, 0]
    i2 = idx2[:, :, 0]
    l1 = jnp.sum(jnp.where(sub2 == i1, lc, 0), axis=1)
    l2 = jnp.sum(jnp.where(sub2 == i2, lc, 0), axis=1)
    none1 = jnp.sum((lc == 1).astype(jnp.int32), axis=1)
    none1 = none1 - (l1 == 1).astype(jnp.int32) - (l2 == 1).astype(jnp.int32)
    v_minus = n_ref[0, 0] - none1
    log_v_minus = jnp.log(v_minus.astype(f32))

    idx1_ref[...] = idx1[:, 0, 0].reshape(1, 1, kb)
    idx2_ref[...] = idx2[:, 0, 0].reshape(1, 1, kb)
    br1_ref[...] = br1.reshape(1, 1, kb)
    br2_ref[...] = br2.reshape(1, 1, kb)
    emb_out_ref[...] = m
    lvp_ref[...] = log_v_plus.reshape(1, 1, kb)
    lvm_ref[...] = log_v_minus.reshape(1, 1, kb)


def kernel(N, leaf_counts_Kxt, embeddings_KxtxD, log, W1, b1, W2, b2):
    k, t, d = embeddings_KxtxD.shape
    h = W1.shape[1]
    kb = KB
    nb = k // kb

    gum = _gumbel_const(k, t)
    n_arr = jnp.asarray(N, jnp.int32).reshape(1, 1)
    leaf_i32 = leaf_counts_Kxt.astype(jnp.int32)
    b1_2d = b1.reshape(1, h)
    b2_2d = b2.reshape(1, d)

    vec_shape = jax.ShapeDtypeStruct((nb, 1, kb), jnp.float32)
    int_shape = jax.ShapeDtypeStruct((nb, 1, kb), jnp.int32)
    out_shapes = (
        int_shape, int_shape, vec_shape, vec_shape,
        jax.ShapeDtypeStruct((k, d), jnp.float32),
        vec_shape, vec_shape,
    )
    vec_spec = pl.BlockSpec((1, 1, kb), lambda i: (i, 0, 0))
    out_specs = (
        vec_spec, vec_spec, vec_spec, vec_spec,
        pl.BlockSpec((kb, d), lambda i: (i, 0)),
        vec_spec, vec_spec,
    )
    in_specs = (
        pl.BlockSpec((1, 1), lambda i: (0, 0)),
        pl.BlockSpec((kb, t, d), lambda i: (i, 0, 0)),
        pl.BlockSpec((kb, t, t), lambda i: (i, 0, 0)),
        pl.BlockSpec((kb, t), lambda i: (i, 0)),
        pl.BlockSpec((2 * d, h), lambda i: (0, 0)),
        pl.BlockSpec((1, h), lambda i: (0, 0)),
        pl.BlockSpec((h, d), lambda i: (0, 0)),
        pl.BlockSpec((1, d), lambda i: (0, 0)),
    )

    outs = pl.pallas_call(
        functools.partial(_proposal_kernel, kb=kb, t=t, d=d),
        grid=(nb,),
        in_specs=in_specs,
        out_specs=out_specs,
        out_shape=out_shapes,
        compiler_params=pltpu.CompilerParams(
            dimension_semantics=("arbitrary",)),
    )(n_arr, embeddings_KxtxD, gum, leaf_i32, W1, b1_2d, W2, b2_2d)

    idx1, idx2, br1, br2, emb_out, lvp, lvm = outs
    return (idx1.reshape(k), idx2.reshape(k), br1.reshape(k), br2.reshape(k),
            emb_out, lvp.reshape(k), lvm.reshape(k))


# speedup vs baseline: 3.8799x; 1.6599x over previous
import functools

import jax
import jax.numpy as jnp
from jax.experimental import pallas as pl
from jax.experimental.pallas import tpu as pltpu

SAMPLE_MERGE_TEMP = 1.0
KB = 64
GSZ = 8

_GUMBEL_CACHE = {}


def _gumbel_const(k, t, kb):
    if (k, t, kb) not in _GUMBEL_CACHE:
        nb, p = k // kb, kb // 2

        def build():
            g = jax.random.gumbel(jax.random.key(42), (k, t * t), jnp.float32)
            g = g.reshape(nb, 2, p, t, t).transpose(0, 2, 3, 1, 4)
            return g.reshape(nb * p, t, 2 * t)
        try:
            with jax.ensure_compile_time_eval():
                gp = jax.block_until_ready(build())
        except Exception:
            return build()
        _GUMBEL_CACHE[(k, t, kb)] = gp
    return _GUMBEL_CACHE[(k, t, kb)]


def _halves(x, li, axis, kind, t):
    if kind == "max":
        sent = jnp.float32(-jnp.inf)
        lo = jnp.max(jnp.where(li < t, x, sent), axis=axis, keepdims=True)
        hi = jnp.max(jnp.where(li >= t, x, sent), axis=axis, keepdims=True)
    elif kind == "min":
        sent = (jnp.int32(2 * t * t) if x.dtype == jnp.int32
                else jnp.float32(jnp.inf))
        lo = jnp.min(jnp.where(li < t, x, sent), axis=axis, keepdims=True)
        hi = jnp.min(jnp.where(li >= t, x, sent), axis=axis, keepdims=True)
    else:
        zero = jnp.float32(0.0)
        lo = jnp.sum(jnp.where(li < t, x, zero), axis=axis, keepdims=True)
        hi = jnp.sum(jnp.where(li >= t, x, zero), axis=axis, keepdims=True)
    return lo, hi


def _proposal_kernel(n_ref, emb_ref, gum_ref, leaf_ref, w1_ref, b1_ref,
                     w2_ref, b2_ref,
                     idx1_ref, idx2_ref, br1_ref, br2_ref, emb_out_ref,
                     lvp_ref, lvm_ref, *, kb, t, d):
    f32 = jnp.float32
    p = kb // 2
    e = emb_ref[...]

    dots = [jax.lax.dot_general(e[i], e[i], (((1,), (1,)), ((), ())),
                                preferred_element_type=f32)
            for i in range(kb)]
    gp = jnp.stack([jnp.concatenate([dots[i], dots[p + i]], axis=1)
                    for i in range(p)], axis=0)

    sq = jnp.sum(e * e, axis=2)
    li = jax.lax.broadcasted_iota(jnp.int32, (1, 1, 2 * t), 2)
    sq_lo = sq[:p][:, :, None]
    sq_hi = sq[p:][:, :, None]
    sq_row = jnp.where(li < t, sq_lo, sq_hi)
    sq_col = jnp.concatenate(
        [sq[:p].reshape(p, 1, t), sq[p:].reshape(p, 1, t)],
        axis=2)

    d2 = (sq_row + sq_col) - 2.0 * gp
    dist = jnp.sqrt(jnp.maximum(d2, 1e-12))

    row = jax.lax.broadcasted_iota(jnp.int32, (1, t, 2 * t), 1)
    lm = jax.lax.broadcasted_iota(jnp.int32, (1, t, 2 * t), 2) % t
    eyeinf = jnp.where(row == lm, jnp.float32(jnp.inf), 0.0)
    distm = dist + eyeinf

    score = gum_ref[...] - distm
    rowmax = jnp.max(score, axis=1, keepdims=True)
    m_lo, m_hi = _halves(rowmax, li, 2, "max", t)
    m_b = jnp.where(li < t, m_lo, m_hi)
    flatio = row * t + lm
    cand = jnp.where(score == m_b, flatio, 2 * t * t)
    rowmin = jnp.min(cand, axis=1, keepdims=True)
    f_lo, f_hi = _halves(rowmin, li, 2, "min", t)
    flat = jnp.concatenate([f_lo, f_hi], axis=0)
    idx1 = (flat // t)[:, :, 0]
    idx2 = (flat % t)[:, :, 0]

    rowmind = jnp.min(distm, axis=1, keepdims=True)
    d_lo, d_hi = _halves(rowmind, li, 2, "min", t)
    dmin_b = jnp.where(li < t, d_lo, d_hi)
    et = jnp.exp(dmin_b - distm)
    rowsum = jnp.sum(et, axis=1, keepdims=True)
    s_lo, s_hi = _halves(rowsum, li, 2, "sum", t)
    ssum = jnp.concatenate([s_lo, s_hi], axis=0)[:, :, 0]
    dmin = jnp.concatenate([d_lo, d_hi], axis=0)[:, :, 0]
    lse = jnp.log(ssum) - dmin

    c1s, c2s = [], []
    gio = jax.lax.broadcasted_iota(jnp.int32, (GSZ, GSZ * t), 1)
    rbase = jax.lax.broadcasted_iota(jnp.int32, (GSZ, 1), 0) * t
    for g in range(kb // GSZ):
        sl = slice(g * GSZ, (g + 1) * GSZ)
        ef = e[sl].reshape(GSZ * t, d)
        oh1 = (gio == rbase + idx1[sl]).astype(f32)
        oh2 = (gio == rbase + idx2[sl]).astype(f32)
        c1s.append(jax.lax.dot_general(oh1, ef, (((1,), (0,)), ((), ())),
                                       preferred_element_type=f32))
        c2s.append(jax.lax.dot_general(oh2, ef, (((1,), (0,)), ((), ())),
                                       preferred_element_type=f32))
    c1 = jnp.concatenate(c1s, axis=0)
    c2 = jnp.concatenate(c2s, axis=0)

    sel = -jnp.sqrt(jnp.maximum(
        jnp.sum((c1 - c2) ** 2, axis=1, keepdims=True), 1e-12))
    log_v_plus = sel + jnp.log(2.0) - lse

    cat = jnp.concatenate([c1, c2], axis=1)
    h = jnp.dot(cat, w1_ref[...], preferred_element_type=f32) + b1_ref[...]
    h = jnp.maximum(h, 0.0)
    m = jnp.dot(h, w2_ref[...], preferred_element_type=f32) + b2_ref[...]

    br1 = jnp.sqrt(jnp.maximum(
        jnp.sum((c1 - m) ** 2, axis=1, keepdims=True), 1e-12))
    br2 = jnp.sqrt(jnp.maximum(
        jnp.sum((c2 - m) ** 2, axis=1, keepdims=True), 1e-12))

    lc = leaf_ref[...]
    sub2 = jax.lax.broadcasted_iota(jnp.int32, (kb, t), 1)
    l1 = jnp.sum(jnp.where(sub2 == idx1, lc, 0), axis=1, keepdims=True)
    l2 = jnp.sum(jnp.where(sub2 == idx2, lc, 0), axis=1, keepdims=True)
    none1 = jnp.sum((lc == 1).astype(jnp.int32), axis=1, keepdims=True)
    none1 = none1 - (l1 == 1).astype(jnp.int32) - (l2 == 1).astype(jnp.int32)
    v_minus = n_ref[0, 0] - none1
    log_v_minus = jnp.log(v_minus.astype(f32))

    idx1_ref[...] = idx1.reshape(1, 1, kb)
    idx2_ref[...] = idx2.reshape(1, 1, kb)
    br1_ref[...] = br1.reshape(1, 1, kb)
    br2_ref[...] = br2.reshape(1, 1, kb)
    emb_out_ref[...] = m
    lvp_ref[...] = log_v_plus.reshape(1, 1, kb)
    lvm_ref[...] = log_v_minus.reshape(1, 1, kb)


def kernel(N, leaf_counts_Kxt, embeddings_KxtxD, log, W1, b1, W2, b2):
    k, t, d = embeddings_KxtxD.shape
    h = W1.shape[1]
    kb = KB
    nb = k // kb

    gum = _gumbel_const(k, t, kb)
    n_arr = jnp.asarray(N, jnp.int32).reshape(1, 1)
    leaf_i32 = leaf_counts_Kxt.astype(jnp.int32)
    b1_2d = b1.reshape(1, h)
    b2_2d = b2.reshape(1, d)

    vec_shape = jax.ShapeDtypeStruct((nb, 1, kb), jnp.float32)
    int_shape = jax.ShapeDtypeStruct((nb, 1, kb), jnp.int32)
    out_shapes = (
        int_shape, int_shape, vec_shape, vec_shape,
        jax.ShapeDtypeStruct((k, d), jnp.float32),
        vec_shape, vec_shape,
    )
    vec_spec = pl.BlockSpec((1, 1, kb), lambda i: (i, 0, 0))
    out_specs = (
        vec_spec, vec_spec, vec_spec, vec_spec,
        pl.BlockSpec((kb, d), lambda i: (i, 0)),
        vec_spec, vec_spec,
    )
    in_specs = (
        pl.BlockSpec((1, 1), lambda i: (0, 0)),
        pl.BlockSpec((kb, t, d), lambda i: (i, 0, 0)),
        pl.BlockSpec((kb // 2, t, 2 * t), lambda i: (i, 0, 0)),
        pl.BlockSpec((kb, t), lambda i: (i, 0)),
        pl.BlockSpec((2 * d, h), lambda i: (0, 0)),
        pl.BlockSpec((1, h), lambda i: (0, 0)),
        pl.BlockSpec((h, d), lambda i: (0, 0)),
        pl.BlockSpec((1, d), lambda i: (0, 0)),
    )

    outs = pl.pallas_call(
        functools.partial(_proposal_kernel, kb=kb, t=t, d=d),
        grid=(nb,),
        in_specs=in_specs,
        out_specs=out_specs,
        out_shape=out_shapes,
        compiler_params=pltpu.CompilerParams(
            dimension_semantics=("arbitrary",)),
    )(n_arr, embeddings_KxtxD, gum, leaf_i32, W1, b1_2d, W2, b2_2d)

    idx1, idx2, br1, br2, emb_out, lvp, lvm = outs
    return (idx1.reshape(k), idx2.reshape(k), br1.reshape(k), br2.reshape(k),
            emb_out, lvp.reshape(k), lvm.reshape(k))


# speedup vs baseline: 4.6795x; 1.2061x over previous
import functools

import jax
import jax.numpy as jnp
from jax.experimental import pallas as pl
from jax.experimental.pallas import tpu as pltpu

SAMPLE_MERGE_TEMP = 1.0
KB = 128
GSZ = 8

_GUMBEL_CACHE = {}


def _gumbel_const(k, t, kb):
    if (k, t, kb) not in _GUMBEL_CACHE:
        nb, p = k // kb, kb // 2

        def build():
            g = jax.random.gumbel(jax.random.key(42), (k, t * t), jnp.float32)
            g = g.reshape(nb, 2, p, t, t).transpose(0, 2, 3, 1, 4)
            return g.reshape(nb * p, t, 2 * t)
        try:
            with jax.ensure_compile_time_eval():
                gp = jax.block_until_ready(build())
        except Exception:
            return build()
        _GUMBEL_CACHE[(k, t, kb)] = gp
    return _GUMBEL_CACHE[(k, t, kb)]


def _halves(x, li, axis, kind, t):
    if kind == "max":
        sent = jnp.float32(-jnp.inf)
        lo = jnp.max(jnp.where(li < t, x, sent), axis=axis, keepdims=True)
        hi = jnp.max(jnp.where(li >= t, x, sent), axis=axis, keepdims=True)
    elif kind == "min":
        sent = (jnp.int32(2 * t * t) if x.dtype == jnp.int32
                else jnp.float32(jnp.inf))
        lo = jnp.min(jnp.where(li < t, x, sent), axis=axis, keepdims=True)
        hi = jnp.min(jnp.where(li >= t, x, sent), axis=axis, keepdims=True)
    else:
        zero = jnp.float32(0.0)
        lo = jnp.sum(jnp.where(li < t, x, zero), axis=axis, keepdims=True)
        hi = jnp.sum(jnp.where(li >= t, x, zero), axis=axis, keepdims=True)
    return lo, hi


def _proposal_kernel(n_ref, emb1_ref, emb2_ref, gum1_ref, gum2_ref, leaf_ref,
                     w1_ref, b1_ref, w2_ref, b2_ref,
                     idx1_ref, idx2_ref, br1_ref, br2_ref, emb_out_ref,
                     lvp_ref, lvm_ref, *, kb, t, d):
    f32 = jnp.float32
    p = kb // 2
    e = jnp.concatenate([emb1_ref[...], emb2_ref[...]], axis=0)

    dots = [jax.lax.dot_general(e[i], e[i], (((1,), (1,)), ((), ())),
                                preferred_element_type=f32)
            for i in range(kb)]
    gp = jnp.stack([jnp.concatenate([dots[i], dots[p + i]], axis=1)
                    for i in range(p)], axis=0)

    sq = jnp.sum(e * e, axis=2)
    li = jax.lax.broadcasted_iota(jnp.int32, (1, 1, 2 * t), 2)
    sq_lo = sq[:p][:, :, None]
    sq_hi = sq[p:][:, :, None]
    sq_row = jnp.where(li < t, sq_lo, sq_hi)
    sq_col = jnp.concatenate(
        [sq[:p].reshape(p, 1, t), sq[p:].reshape(p, 1, t)],
        axis=2)

    d2 = (sq_row + sq_col) - 2.0 * gp
    dist = jnp.sqrt(jnp.maximum(d2, 1e-12))

    row = jax.lax.broadcasted_iota(jnp.int32, (1, t, 2 * t), 1)
    lm = jax.lax.broadcasted_iota(jnp.int32, (1, t, 2 * t), 2) % t
    eyeinf = jnp.where(row == lm, jnp.float32(jnp.inf), 0.0)
    distm = dist + eyeinf

    gum = jnp.concatenate([gum1_ref[...], gum2_ref[...]], axis=0)
    score = gum - distm
    rowmax = jnp.max(score, axis=1, keepdims=True)
    m_lo, m_hi = _halves(rowmax, li, 2, "max", t)
    m_b = jnp.where(li < t, m_lo, m_hi)
    flatio = row * t + lm
    cand = jnp.where(score == m_b, flatio, 2 * t * t)
    rowmin = jnp.min(cand, axis=1, keepdims=True)
    f_lo, f_hi = _halves(rowmin, li, 2, "min", t)
    flat = jnp.concatenate([f_lo, f_hi], axis=0)
    idx1 = (flat // t)[:, :, 0]
    idx2 = (flat % t)[:, :, 0]

    rowmind = jnp.min(distm, axis=1, keepdims=True)
    d_lo, d_hi = _halves(rowmind, li, 2, "min", t)
    dmin_b = jnp.where(li < t, d_lo, d_hi)
    et = jnp.exp(dmin_b - distm)
    rowsum = jnp.sum(et, axis=1, keepdims=True)
    s_lo, s_hi = _halves(rowsum, li, 2, "sum", t)
    ssum = jnp.concatenate([s_lo, s_hi], axis=0)[:, :, 0]
    dmin = jnp.concatenate([d_lo, d_hi], axis=0)[:, :, 0]
    lse = jnp.log(ssum) - dmin

    c1s, c2s = [], []
    gio = jax.lax.broadcasted_iota(jnp.int32, (GSZ, GSZ * t), 1)
    rbase = jax.lax.broadcasted_iota(jnp.int32, (GSZ, 1), 0) * t
    for g in range(kb // GSZ):
        sl = slice(g * GSZ, (g + 1) * GSZ)
        ef = e[sl].reshape(GSZ * t, d)
        oh1 = (gio == rbase + idx1[sl]).astype(f32)
        oh2 = (gio == rbase + idx2[sl]).astype(f32)
        c1s.append(jax.lax.dot_general(oh1, ef, (((1,), (0,)), ((), ())),
                                       preferred_element_type=f32))
        c2s.append(jax.lax.dot_general(oh2, ef, (((1,), (0,)), ((), ())),
                                       preferred_element_type=f32))
    c1 = jnp.concatenate(c1s, axis=0)
    c2 = jnp.concatenate(c2s, axis=0)

    sel = -jnp.sqrt(jnp.maximum(
        jnp.sum((c1 - c2) ** 2, axis=1, keepdims=True), 1e-12))
    log_v_plus = sel + jnp.log(2.0) - lse

    cat = jnp.concatenate([c1, c2], axis=1)
    h = jnp.dot(cat, w1_ref[...], preferred_element_type=f32) + b1_ref[...]
    h = jnp.maximum(h, 0.0)
    m = jnp.dot(h, w2_ref[...], preferred_element_type=f32) + b2_ref[...]

    br1 = jnp.sqrt(jnp.maximum(
        jnp.sum((c1 - m) ** 2, axis=1, keepdims=True), 1e-12))
    br2 = jnp.sqrt(jnp.maximum(
        jnp.sum((c2 - m) ** 2, axis=1, keepdims=True), 1e-12))

    lc = leaf_ref[...]
    sub2 = jax.lax.broadcasted_iota(jnp.int32, (kb, t), 1)
    l1 = jnp.sum(jnp.where(sub2 == idx1, lc, 0), axis=1, keepdims=True)
    l2 = jnp.sum(jnp.where(sub2 == idx2, lc, 0), axis=1, keepdims=True)
    none1 = jnp.sum((lc == 1).astype(jnp.int32), axis=1, keepdims=True)
    none1 = none1 - (l1 == 1).astype(jnp.int32) - (l2 == 1).astype(jnp.int32)
    v_minus = n_ref[0, 0] - none1
    log_v_minus = jnp.log(v_minus.astype(f32))

    idx1_ref[...] = idx1.reshape(1, 1, kb)
    idx2_ref[...] = idx2.reshape(1, 1, kb)
    br1_ref[...] = br1.reshape(1, 1, kb)
    br2_ref[...] = br2.reshape(1, 1, kb)
    emb_out_ref[...] = m
    lvp_ref[...] = log_v_plus.reshape(1, 1, kb)
    lvm_ref[...] = log_v_minus.reshape(1, 1, kb)


def kernel(N, leaf_counts_Kxt, embeddings_KxtxD, log, W1, b1, W2, b2):
    k, t, d = embeddings_KxtxD.shape
    h = W1.shape[1]
    kb = KB
    nb = k // kb

    gum = _gumbel_const(k, t, kb)
    n_arr = jnp.asarray(N, jnp.int32).reshape(1, 1)
    leaf_i32 = leaf_counts_Kxt.astype(jnp.int32)
    b1_2d = b1.reshape(1, h)
    b2_2d = b2.reshape(1, d)

    vec_shape = jax.ShapeDtypeStruct((nb, 1, kb), jnp.float32)
    int_shape = jax.ShapeDtypeStruct((nb, 1, kb), jnp.int32)
    out_shapes = (
        int_shape, int_shape, vec_shape, vec_shape,
        jax.ShapeDtypeStruct((k, d), jnp.float32),
        vec_shape, vec_shape,
    )
    vec_spec = pl.BlockSpec((1, 1, kb), lambda i: (i, 0, 0))
    out_specs = (
        vec_spec, vec_spec, vec_spec, vec_spec,
        pl.BlockSpec((kb, d), lambda i: (i, 0)),
        vec_spec, vec_spec,
    )
    in_specs = (
        pl.BlockSpec((1, 1), lambda i: (0, 0)),
        pl.BlockSpec((kb // 2, t, d), lambda i: (2 * i, 0, 0)),
        pl.BlockSpec((kb // 2, t, d), lambda i: (2 * i + 1, 0, 0)),
        pl.BlockSpec((kb // 4, t, 2 * t), lambda i: (2 * i, 0, 0)),
        pl.BlockSpec((kb // 4, t, 2 * t), lambda i: (2 * i + 1, 0, 0)),
        pl.BlockSpec((kb, t), lambda i: (i, 0)),
        pl.BlockSpec((2 * d, h), lambda i: (0, 0)),
        pl.BlockSpec((1, h), lambda i: (0, 0)),
        pl.BlockSpec((h, d), lambda i: (0, 0)),
        pl.BlockSpec((1, d), lambda i: (0, 0)),
    )

    outs = pl.pallas_call(
        functools.partial(_proposal_kernel, kb=kb, t=t, d=d),
        grid=(nb,),
        in_specs=in_specs,
        out_specs=out_specs,
        out_shape=out_shapes,
        compiler_params=pltpu.CompilerParams(
            dimension_semantics=("arbitrary",)),
    )(n_arr, embeddings_KxtxD, embeddings_KxtxD, gum, gum, leaf_i32,
      W1, b1_2d, W2, b2_2d)

    idx1, idx2, br1, br2, emb_out, lvp, lvm = outs
    return (idx1.reshape(k), idx2.reshape(k), br1.reshape(k), br2.reshape(k),
            emb_out, lvp.reshape(k), lvm.reshape(k))


# speedup vs baseline: 4.6994x; 1.0043x over previous
import functools

import jax
import jax.numpy as jnp
from jax.experimental import pallas as pl
from jax.experimental.pallas import tpu as pltpu

SAMPLE_MERGE_TEMP = 1.0
KB = 128
GSZ = 8

_GUMBEL_CACHE = {}


def _gumbel_const(k, t, kb):
    if (k, t, kb) not in _GUMBEL_CACHE:
        nb, p = k // kb, kb // 2

        def build():
            g = jax.random.gumbel(jax.random.key(42), (k, t * t), jnp.float32)
            g = g.reshape(nb, 2, p, t, t).transpose(0, 2, 3, 1, 4)
            return g.reshape(nb * p, t, 2 * t)
        try:
            with jax.ensure_compile_time_eval():
                gp = jax.block_until_ready(build())
        except Exception:
            return build()
        _GUMBEL_CACHE[(k, t, kb)] = gp
    return _GUMBEL_CACHE[(k, t, kb)]


def _halves(x, li, axis, kind, t):
    if kind == "max":
        sent = jnp.float32(-jnp.inf)
        lo = jnp.max(jnp.where(li < t, x, sent), axis=axis, keepdims=True)
        hi = jnp.max(jnp.where(li >= t, x, sent), axis=axis, keepdims=True)
    elif kind == "min":
        sent = (jnp.int32(2 * t * t) if x.dtype == jnp.int32
                else jnp.float32(jnp.inf))
        lo = jnp.min(jnp.where(li < t, x, sent), axis=axis, keepdims=True)
        hi = jnp.min(jnp.where(li >= t, x, sent), axis=axis, keepdims=True)
    else:
        zero = jnp.float32(0.0)
        lo = jnp.sum(jnp.where(li < t, x, zero), axis=axis, keepdims=True)
        hi = jnp.sum(jnp.where(li >= t, x, zero), axis=axis, keepdims=True)
    return lo, hi


def _proposal_kernel(n_ref, emb1_ref, emb2_ref, gum1_ref, gum2_ref, leaf_ref,
                     w1_ref, b1_ref, w2_ref, b2_ref,
                     idx1_ref, idx2_ref, br1_ref, br2_ref, emb_out_ref,
                     lvp_ref, lvm_ref, *, kb, t, d):
    f32 = jnp.float32
    p = kb // 2
    e1 = emb1_ref[...]
    e2 = emb2_ref[...]

    def epart(i):
        return e1[i] if i < p else e2[i - p]

    dots = [jax.lax.dot_general(epart(i), epart(i), (((1,), (1,)), ((), ())),
                                preferred_element_type=f32)
            for i in range(kb)]
    gp = jnp.stack([jnp.concatenate([dots[i], dots[p + i]], axis=1)
                    for i in range(p)], axis=0)

    sq1 = jnp.sum(e1 * e1, axis=2)
    sq2 = jnp.sum(e2 * e2, axis=2)
    li = jax.lax.broadcasted_iota(jnp.int32, (1, 1, 2 * t), 2)
    sq_lo = sq1[:, :, None]
    sq_hi = sq2[:, :, None]
    sq_row = jnp.where(li < t, sq_lo, sq_hi)
    sq_col = jnp.concatenate(
        [sq1.reshape(p, 1, t), sq2.reshape(p, 1, t)],
        axis=2)

    d2 = (sq_row + sq_col) - 2.0 * gp
    dist = jnp.sqrt(jnp.maximum(d2, 1e-12))

    row = jax.lax.broadcasted_iota(jnp.int32, (1, t, 2 * t), 1)
    lm = jax.lax.broadcasted_iota(jnp.int32, (1, t, 2 * t), 2) % t
    eyeinf = jnp.where(row == lm, jnp.float32(jnp.inf), 0.0)
    distm = dist + eyeinf

    gum = jnp.concatenate([gum1_ref[...], gum2_ref[...]], axis=0)
    score = gum - distm
    rowmax = jnp.max(score, axis=1, keepdims=True)
    m_lo, m_hi = _halves(rowmax, li, 2, "max", t)
    m_b = jnp.where(li < t, m_lo, m_hi)
    flatio = row * t + lm
    cand = jnp.where(score == m_b, flatio, 2 * t * t)
    rowmin = jnp.min(cand, axis=1, keepdims=True)
    f_lo, f_hi = _halves(rowmin, li, 2, "min", t)
    flat = jnp.concatenate([f_lo, f_hi], axis=0)
    idx1 = (flat // t)[:, :, 0]
    idx2 = (flat % t)[:, :, 0]

    rowmind = jnp.min(distm, axis=1, keepdims=True)
    d_lo, d_hi = _halves(rowmind, li, 2, "min", t)
    dmin_b = jnp.where(li < t, d_lo, d_hi)
    et = jnp.exp(dmin_b - distm)
    rowsum = jnp.sum(et, axis=1, keepdims=True)
    s_lo, s_hi = _halves(rowsum, li, 2, "sum", t)
    ssum = jnp.concatenate([s_lo, s_hi], axis=0)[:, :, 0]
    dmin = jnp.concatenate([d_lo, d_hi], axis=0)[:, :, 0]
    lse = jnp.log(ssum) - dmin

    c1s, c2s = [], []
    gio = jax.lax.broadcasted_iota(jnp.int32, (GSZ, GSZ * t), 1)
    rbase = jax.lax.broadcasted_iota(jnp.int32, (GSZ, 1), 0) * t
    ngh = p // GSZ
    for g in range(kb // GSZ):
        sl = slice(g * GSZ, (g + 1) * GSZ)
        eh = (e1[sl] if g < ngh
              else e2[(g - ngh) * GSZ:(g - ngh + 1) * GSZ])
        ef = eh.reshape(GSZ * t, d)
        oh1 = (gio == rbase + idx1[sl]).astype(f32)
        oh2 = (gio == rbase + idx2[sl]).astype(f32)
        c1s.append(jax.lax.dot_general(oh1, ef, (((1,), (0,)), ((), ())),
                                       preferred_element_type=f32))
        c2s.append(jax.lax.dot_general(oh2, ef, (((1,), (0,)), ((), ())),
                                       preferred_element_type=f32))
    c1 = jnp.concatenate(c1s, axis=0)
    c2 = jnp.concatenate(c2s, axis=0)

    sel = -jnp.sqrt(jnp.maximum(
        jnp.sum((c1 - c2) ** 2, axis=1, keepdims=True), 1e-12))
    log_v_plus = sel + jnp.log(2.0) - lse

    cat = jnp.concatenate([c1, c2], axis=1)
    h = jnp.dot(cat, w1_ref[...], preferred_element_type=f32) + b1_ref[...]
    h = jnp.maximum(h, 0.0)
    m = jnp.dot(h, w2_ref[...], preferred_element_type=f32) + b2_ref[...]

    br1 = jnp.sqrt(jnp.maximum(
        jnp.sum((c1 - m) ** 2, axis=1, keepdims=True), 1e-12))
    br2 = jnp.sqrt(jnp.maximum(
        jnp.sum((c2 - m) ** 2, axis=1, keepdims=True), 1e-12))

    lc = leaf_ref[...]
    sub2 = jax.lax.broadcasted_iota(jnp.int32, (kb, t), 1)
    l1 = jnp.sum(jnp.where(sub2 == idx1, lc, 0), axis=1, keepdims=True)
    l2 = jnp.sum(jnp.where(sub2 == idx2, lc, 0), axis=1, keepdims=True)
    none1 = jnp.sum((lc == 1).astype(jnp.int32), axis=1, keepdims=True)
    none1 = none1 - (l1 == 1).astype(jnp.int32) - (l2 == 1).astype(jnp.int32)
    v_minus = n_ref[0, 0] - none1
    log_v_minus = jnp.log(v_minus.astype(f32))

    idx1_ref[...] = idx1.reshape(1, 1, kb)
    idx2_ref[...] = idx2.reshape(1, 1, kb)
    br1_ref[...] = br1.reshape(1, 1, kb)
    br2_ref[...] = br2.reshape(1, 1, kb)
    emb_out_ref[...] = m
    lvp_ref[...] = log_v_plus.reshape(1, 1, kb)
    lvm_ref[...] = log_v_minus.reshape(1, 1, kb)


def kernel(N, leaf_counts_Kxt, embeddings_KxtxD, log, W1, b1, W2, b2):
    k, t, d = embeddings_KxtxD.shape
    h = W1.shape[1]
    kb = KB
    nb = k // kb

    gum = _gumbel_const(k, t, kb)
    n_arr = jnp.asarray(N, jnp.int32).reshape(1, 1)
    leaf_i32 = leaf_counts_Kxt.astype(jnp.int32)
    b1_2d = b1.reshape(1, h)
    b2_2d = b2.reshape(1, d)

    vec_shape = jax.ShapeDtypeStruct((nb, 1, kb), jnp.float32)
    int_shape = jax.ShapeDtypeStruct((nb, 1, kb), jnp.int32)
    out_shapes = (
        int_shape, int_shape, vec_shape, vec_shape,
        jax.ShapeDtypeStruct((k, d), jnp.float32),
        vec_shape, vec_shape,
    )
    vec_spec = pl.BlockSpec((1, 1, kb), lambda i: (i, 0, 0))
    out_specs = (
        vec_spec, vec_spec, vec_spec, vec_spec,
        pl.BlockSpec((kb, d), lambda i: (i, 0)),
        vec_spec, vec_spec,
    )
    in_specs = (
        pl.BlockSpec((1, 1), lambda i: (0, 0)),
        pl.BlockSpec((kb // 2, t, d), lambda i: (2 * i, 0, 0)),
        pl.BlockSpec((kb // 2, t, d), lambda i: (2 * i + 1, 0, 0)),
        pl.BlockSpec((kb // 4, t, 2 * t), lambda i: (2 * i, 0, 0)),
        pl.BlockSpec((kb // 4, t, 2 * t), lambda i: (2 * i + 1, 0, 0)),
        pl.BlockSpec((kb, t), lambda i: (i, 0)),
        pl.BlockSpec((2 * d, h), lambda i: (0, 0)),
        pl.BlockSpec((1, h), lambda i: (0, 0)),
        pl.BlockSpec((h, d), lambda i: (0, 0)),
        pl.BlockSpec((1, d), lambda i: (0, 0)),
    )

    outs = pl.pallas_call(
        functools.partial(_proposal_kernel, kb=kb, t=t, d=d),
        grid=(nb,),
        in_specs=in_specs,
        out_specs=out_specs,
        out_shape=out_shapes,
        compiler_params=pltpu.CompilerParams(
            dimension_semantics=("arbitrary",)),
    )(n_arr, embeddings_KxtxD, embeddings_KxtxD, gum, gum, leaf_i32,
      W1, b1_2d, W2, b2_2d)

    idx1, idx2, br1, br2, emb_out, lvp, lvm = outs
    return (idx1.reshape(k), idx2.reshape(k), br1.reshape(k), br2.reshape(k),
            emb_out, lvp.reshape(k), lvm.reshape(k))
